# baseline (speedup 1.0000x reference)
import dataclasses
import functools

import jax
import jax.numpy as jnp
from jax import lax
from jax.experimental import pallas as pl
from jax.experimental.pallas import tpu as pltpu
from jax.experimental.pallas import tpu_sc as plsc

NC = 2
NS = 16
NT = NC * NS
CHUNK = 128


def _cdiv(a, b):
    return (a + b - 1) // b


def _mesh():
    return plsc.VectorSubcoreMesh(
        core_axis_name="c", subcore_axis_name="s", num_cores=NC, num_subcores=NS
    )


@functools.lru_cache(maxsize=None)
def _deg_kernel(K, ZR):
    cp = pltpu.CompilerParams()
    if "needs_layout_passes" in pltpu.CompilerParams.__dataclass_fields__:
        cp = dataclasses.replace(cp, needs_layout_passes=False)

    @functools.partial(
        pl.kernel,
        out_type=jax.ShapeDtypeStruct((NT, ZR), jnp.float32),
        mesh=_mesh(),
        compiler_params=cp,
        scratch_types=[
            pltpu.VMEM((K, CHUNK), jnp.int32),
            pltpu.VMEM((ZR,), jnp.float32),
        ],
    )
    def deg(dst_hbm, out_hbm, idx_v, deg_v):
        c = lax.axis_index("c")
        s = lax.axis_index("s")
        tile = s * NC + c

        @pl.loop(0, ZR, step=16)
        def _(i):
            deg_v[pl.ds(i, 16)] = jnp.zeros((16,), jnp.float32)

        pltpu.sync_copy(dst_hbm.at[pl.ds(tile * K, K)], idx_v)
        ones = jnp.ones((16,), jnp.float32)

        @pl.loop(0, K)
        def _(j):
            @pl.loop(0, CHUNK, step=16)
            def _(k):
                idx16 = idx_v[j, pl.ds(k, 16)]
                plsc.addupdate_scatter(deg_v, [idx16], ones)

        pltpu.sync_copy(deg_v, out_hbm.at[tile])

    return deg


IG = 16


@functools.lru_cache(maxsize=None)
def _scatter_kernel(K, ZR, N, D):
    RPT = ZR // NS

    @functools.partial(
        pl.kernel,
        out_type=jax.ShapeDtypeStruct((NC, ZR, D), jnp.float32),
        mesh=_mesh(),
        scratch_types=[
            pltpu.VMEM((IG, CHUNK), jnp.int32),
            pltpu.VMEM((IG, CHUNK), jnp.int32),
            pltpu.VMEM((CHUNK, D), jnp.float32),
            pltpu.VMEM((CHUNK, D), jnp.float32),
            pltpu.SemaphoreType.DMA,
            pltpu.SemaphoreType.DMA,
            pltpu.SemaphoreType.DMA,
            pltpu.SemaphoreType.DMA,
            pltpu.VMEM_SHARED((ZR, D), jnp.float32),
        ],
    )
    def scat(y_hbm, src_hbm, dst_hbm, zeros_hbm, out_hbm,
             si_v, di_v, big0, big1, gs0, gs1, ss0, ss1, z_sh):
        big = (big0, big1)
        gsem = (gs0, gs1)
        ssem = (ss0, ss1)
        c = lax.axis_index("c")
        s = lax.axis_index("s")
        tile = s * NC + c

        pltpu.sync_copy(zeros_hbm, z_sh.at[pl.ds(s * RPT, RPT)])
        plsc.subcore_barrier()

        @pl.loop(0, K, step=IG)
        def _(g):
            base = tile * K + g
            pltpu.sync_copy(src_hbm.at[pl.ds(base, IG)], si_v)
            pltpu.sync_copy(dst_hbm.at[pl.ds(base, IG)], di_v)
            gd = {}
            sd = {}

            def gather(u, b):
                gd[b] = pltpu.async_copy(
                    y_hbm.at[si_v.at[u]], big[b], gsem[b]
                )

            def scatter(u, b):
                sd[b] = pltpu.async_copy(
                    big[b], z_sh.at[di_v.at[u]], ssem[b], add=True
                )

            for u in range(IG):
                b = u & 1
                if b in sd:
                    sd.pop(b).wait()
                gather(u, b)
                if u >= 1:
                    bb = (u - 1) & 1
                    gd.pop(bb).wait()
                    scatter(u - 1, bb)
            bb = (IG - 1) & 1
            gd.pop(bb).wait()
            scatter(IG - 1, bb)
            for b in (0, 1):
                if b in sd:
                    sd.pop(b).wait()

        plsc.subcore_barrier()
        pltpu.sync_copy(
            z_sh.at[pl.ds(s * RPT, RPT)], out_hbm.at[c, pl.ds(s * RPT, RPT)]
        )

    return scat


def _r_body(dp_ref, o_ref):
    o_ref[...] = lax.rsqrt(jnp.sum(dp_ref[...], axis=0, keepdims=True) + 1.0)


def _a_body(x_ref, w_ref, dis_ref, o_ref):
    xw = jnp.dot(x_ref[...], w_ref[...], preferred_element_type=jnp.float32)
    o_ref[...] = xw * dis_ref[...]


def _norm(z0, z1, y, dis, b, g, be):
    pre = (z0 + z1 + y) * dis + b
    mu = jnp.mean(pre, axis=0, keepdims=True)
    ctr = pre - mu
    var = jnp.mean(ctr * ctr, axis=0, keepdims=True)
    return jnp.maximum(ctr * lax.rsqrt(var + 1e-5) * g + be, 0.0)


def _f_body(z0_ref, z1_ref, y_ref, dis_ref, b_ref, g_ref, be_ref, w_ref, o_ref):
    h = _norm(z0_ref[...], z1_ref[...], y_ref[...], dis_ref[...], b_ref[...],
              g_ref[...], be_ref[...])
    xw = jnp.dot(h, w_ref[...], preferred_element_type=jnp.float32)
    o_ref[...] = xw * dis_ref[...]


def _g_body(z0_ref, z1_ref, y_ref, dis_ref, b_ref, g_ref, be_ref, h_ref, ge_ref):
    h = _norm(z0_ref[...], z1_ref[...], y_ref[...], dis_ref[...], b_ref[...],
              g_ref[...], be_ref[...])
    h_ref[...] = h
    ge_ref[...] = jnp.mean(h, axis=0, keepdims=True)


def kernel(x, edge_index, W0, b0, g0, be0, W1, b1, g1, be1, W2, b2, g2, be2):
    N, D = x.shape
    E = edge_index.shape[1]
    CH = _cdiv(_cdiv(E, CHUNK), NT * IG) * NT * IG
    K = CH // NT
    E_pad = CH * CHUNK
    ZR = _cdiv(N + 1, NS * 8) * NS * 8
    RPT = ZR // NS
    dummy = jnp.int32(N)

    src = edge_index[0]
    dst = edge_index[1]
    pad = E_pad - E
    src_p = jnp.concatenate([src, jnp.zeros((pad,), src.dtype)]).reshape(CH, CHUNK)
    dst_p = jnp.concatenate([dst, jnp.full((pad,), dummy, dst.dtype)]).reshape(
        CH, CHUNK
    )
    zeros_t = jnp.zeros((RPT, D), jnp.float32)

    degparts = _deg_kernel(K, ZR)(dst_p)
    dis_row = pl.pallas_call(
        _r_body, out_shape=jax.ShapeDtypeStruct((1, ZR), jnp.float32)
    )(degparts)
    dis_col = dis_row[0, :N].reshape(N, 1)

    f32 = jnp.float32
    nd = jax.ShapeDtypeStruct((N, D), f32)
    row = jax.ShapeDtypeStruct((1, D), f32)
    scat = _scatter_kernel(K, ZR, N, D)
    b0r, g0r, be0r = b0.reshape(1, D), g0.reshape(1, D), be0.reshape(1, D)
    b1r, g1r, be1r = b1.reshape(1, D), g1.reshape(1, D), be1.reshape(1, D)
    b2r, g2r, be2r = b2.reshape(1, D), g2.reshape(1, D), be2.reshape(1, D)

    y = pl.pallas_call(_a_body, out_shape=nd)(x, W0, dis_col)

    zraw = scat(y, src_p, dst_p, zeros_t)
    y = pl.pallas_call(_f_body, out_shape=nd)(
        zraw[0, :N], zraw[1, :N], y, dis_col, b0r, g0r, be0r, W1
    )

    zraw = scat(y, src_p, dst_p, zeros_t)
    y = pl.pallas_call(_f_body, out_shape=nd)(
        zraw[0, :N], zraw[1, :N], y, dis_col, b1r, g1r, be1r, W2
    )

    zraw = scat(y, src_p, dst_p, zeros_t)
    h, ge = pl.pallas_call(_g_body, out_shape=(nd, row))(
        zraw[0, :N], zraw[1, :N], y, dis_col, b2r, g2r, be2r
    )
    return h, ge


# speedup vs baseline: 1.4460x; 1.4460x over previous
import dataclasses
import functools
import math

import jax
import jax.numpy as jnp
from jax import lax
from jax.experimental import pallas as pl
from jax.experimental.pallas import tpu as pltpu
from jax.experimental.pallas import tpu_sc as plsc

NC = 2
NS = 16
NT = NC * NS
CHUNK = 128


def _cdiv(a, b):
    return (a + b - 1) // b


def _mesh():
    return plsc.VectorSubcoreMesh(
        core_axis_name="c", subcore_axis_name="s", num_cores=NC, num_subcores=NS
    )


@functools.lru_cache(maxsize=None)
def _deg_kernel(K, ZR):
    cp = pltpu.CompilerParams()
    if "needs_layout_passes" in pltpu.CompilerParams.__dataclass_fields__:
        cp = dataclasses.replace(cp, needs_layout_passes=False)

    @functools.partial(
        pl.kernel,
        out_type=jax.ShapeDtypeStruct((NT, ZR), jnp.float32),
        mesh=_mesh(),
        compiler_params=cp,
        scratch_types=[
            pltpu.VMEM((K, CHUNK), jnp.int32),
            pltpu.VMEM((ZR,), jnp.float32),
        ],
    )
    def deg(dst_hbm, out_hbm, idx_v, deg_v):
        c = lax.axis_index("c")
        s = lax.axis_index("s")
        tile = s * NC + c

        @pl.loop(0, ZR, step=16)
        def _(i):
            deg_v[pl.ds(i, 16)] = jnp.zeros((16,), jnp.float32)

        pltpu.sync_copy(dst_hbm.at[pl.ds(tile * K, K)], idx_v)
        ones = jnp.ones((16,), jnp.float32)

        @pl.loop(0, K)
        def _(j):
            @pl.loop(0, CHUNK, step=16)
            def _(k):
                idx16 = idx_v[j, pl.ds(k, 16)]
                plsc.addupdate_scatter(deg_v, [idx16], ones)

        pltpu.sync_copy(deg_v, out_hbm.at[tile])

    return deg


SS = 4
IG = 40


@functools.lru_cache(maxsize=None)
def _scatter_kernel(K2, ZR, N, D):
    RPT = ZR // NS
    Dh = D // 2

    @functools.partial(
        pl.kernel,
        out_type=jax.ShapeDtypeStruct((NC, ZR, Dh), jnp.float32),
        mesh=_mesh(),
        compiler_params=pltpu.CompilerParams(use_tc_tiling_on_sc=False),
        scratch_types=[
            pltpu.VMEM((IG, CHUNK), jnp.int32),
            pltpu.VMEM((IG, CHUNK), jnp.int32),
            pltpu.VMEM((SS * CHUNK, Dh), jnp.float32),
            pltpu.VMEM((SS * CHUNK, Dh), jnp.float32),
            pltpu.SemaphoreType.DMA,
            pltpu.SemaphoreType.DMA,
            pltpu.SemaphoreType.DMA,
            pltpu.SemaphoreType.DMA,
            pltpu.VMEM_SHARED((ZR, Dh), jnp.float32),
        ],
    )
    def scat(y2_hbm, src_hbm, dst_hbm, zeros_hbm, out_hbm,
             si_v, di_v, big0, big1, gs0, gs1, ss0, ss1, z_sh):
        big = (big0, big1)
        gsem = (gs0, gs1)
        ssem = (ss0, ss1)
        c = lax.axis_index("c")
        s = lax.axis_index("s")
        ysrc = y2_hbm.at[c]

        pltpu.sync_copy(zeros_hbm, z_sh.at[pl.ds(s * RPT, RPT)])
        plsc.subcore_barrier()

        nsup = IG // SS

        def issue_gathers(u, b):
            return [
                pltpu.async_copy(
                    ysrc.at[si_v.at[u * SS + k]],
                    big[b].at[pl.ds(k * CHUNK, CHUNK)],
                    gsem[b],
                )
                for k in range(SS)
            ]

        def issue_scatters(u, b):
            return [
                pltpu.async_copy(
                    big[b].at[pl.ds(k * CHUNK, CHUNK)],
                    z_sh.at[di_v.at[u * SS + k]],
                    ssem[b],
                    add=True,
                )
                for k in range(SS)
            ]

        @pl.loop(0, K2, step=IG)
        def _(g):
            base = s * K2 + g
            pltpu.sync_copy(src_hbm.at[pl.ds(base, IG)], si_v)
            pltpu.sync_copy(dst_hbm.at[pl.ds(base, IG)], di_v)
            gd = {}
            sd = {}
            for u in range(nsup):
                b = u & 1
                if b in sd:
                    for dsc in sd.pop(b):
                        dsc.wait()
                gd[b] = issue_gathers(u, b)
                if u >= 1:
                    bb = (u - 1) & 1
                    for dsc in gd.pop(bb):
                        dsc.wait()
                    sd[bb] = issue_scatters(u - 1, bb)
            bb = (nsup - 1) & 1
            for dsc in gd.pop(bb):
                dsc.wait()
            sd[bb] = issue_scatters(nsup - 1, bb)
            for b in (0, 1):
                if b in sd:
                    for dsc in sd.pop(b):
                        dsc.wait()

        plsc.subcore_barrier()
        pltpu.sync_copy(
            z_sh.at[pl.ds(s * RPT, RPT)], out_hbm.at[c, pl.ds(s * RPT, RPT)]
        )

    return scat


def _r_body(dp_ref, o_ref):
    o_ref[...] = lax.rsqrt(jnp.sum(dp_ref[...], axis=0, keepdims=True) + 1.0)


def _split(xw, o_ref):
    dh = xw.shape[1] // 2
    o_ref[0] = xw[:, :dh]
    o_ref[1] = xw[:, dh:]


def _a_body(x_ref, w_ref, dis_ref, o_ref):
    xw = jnp.dot(x_ref[...], w_ref[...], preferred_element_type=jnp.float32)
    _split(xw * dis_ref[...], o_ref)


def _norm(z0, z1, y2, dis, b, g, be):
    pre = jnp.concatenate([z0 + y2[0], z1 + y2[1]], axis=1) * dis + b
    mu = jnp.mean(pre, axis=0, keepdims=True)
    ctr = pre - mu
    var = jnp.mean(ctr * ctr, axis=0, keepdims=True)
    return jnp.maximum(ctr * lax.rsqrt(var + 1e-5) * g + be, 0.0)


def _f_body(z0_ref, z1_ref, y_ref, dis_ref, b_ref, g_ref, be_ref, w_ref, o_ref):
    h = _norm(z0_ref[...], z1_ref[...], y_ref[...], dis_ref[...], b_ref[...],
              g_ref[...], be_ref[...])
    xw = jnp.dot(h, w_ref[...], preferred_element_type=jnp.float32)
    _split(xw * dis_ref[...], o_ref)


def _g_body(z0_ref, z1_ref, y_ref, dis_ref, b_ref, g_ref, be_ref, h_ref, ge_ref):
    h = _norm(z0_ref[...], z1_ref[...], y_ref[...], dis_ref[...], b_ref[...],
              g_ref[...], be_ref[...])
    h_ref[...] = h
    ge_ref[...] = jnp.mean(h, axis=0, keepdims=True)


def kernel(x, edge_index, W0, b0, g0, be0, W1, b1, g1, be1, W2, b2, g2, be2):
    N, D = x.shape
    Dh = D // 2
    E = edge_index.shape[1]
    chm = math.lcm(NS * IG, NT * 8)
    CH = _cdiv(_cdiv(E, CHUNK), chm) * chm
    K2 = CH // NS
    E_pad = CH * CHUNK
    ZR = _cdiv(N + 1, NS * 8) * NS * 8
    RPT = ZR // NS
    dummy = jnp.int32(N)

    src = edge_index[0]
    dst = edge_index[1]
    pad = E_pad - E
    src_p = jnp.concatenate([src, jnp.zeros((pad,), src.dtype)]).reshape(CH, CHUNK)
    dst_p = jnp.concatenate([dst, jnp.full((pad,), dummy, dst.dtype)]).reshape(
        CH, CHUNK
    )
    zeros_t = jnp.zeros((RPT, Dh), jnp.float32)

    degparts = _deg_kernel(CH // NT, ZR)(dst_p)
    dis_row = pl.pallas_call(
        _r_body, out_shape=jax.ShapeDtypeStruct((1, ZR), jnp.float32)
    )(degparts)
    dis_col = dis_row[0, :N].reshape(N, 1)

    f32 = jnp.float32
    nd = jax.ShapeDtypeStruct((N, D), f32)
    nd2 = jax.ShapeDtypeStruct((NC, N, Dh), f32)
    row = jax.ShapeDtypeStruct((1, D), f32)
    scat = _scatter_kernel(K2, ZR, N, D)
    b0r, g0r, be0r = b0.reshape(1, D), g0.reshape(1, D), be0.reshape(1, D)
    b1r, g1r, be1r = b1.reshape(1, D), g1.reshape(1, D), be1.reshape(1, D)
    b2r, g2r, be2r = b2.reshape(1, D), g2.reshape(1, D), be2.reshape(1, D)

    y2 = pl.pallas_call(_a_body, out_shape=nd2)(x, W0, dis_col)

    zraw = scat(y2, src_p, dst_p, zeros_t)
    y2 = pl.pallas_call(_f_body, out_shape=nd2)(
        zraw[0, :N], zraw[1, :N], y2, dis_col, b0r, g0r, be0r, W1
    )

    zraw = scat(y2, src_p, dst_p, zeros_t)
    y2 = pl.pallas_call(_f_body, out_shape=nd2)(
        zraw[0, :N], zraw[1, :N], y2, dis_col, b1r, g1r, be1r, W2
    )

    zraw = scat(y2, src_p, dst_p, zeros_t)
    h, ge = pl.pallas_call(_g_body, out_shape=(nd, row))(
        zraw[0, :N], zraw[1, :N], y2, dis_col, b2r, g2r, be2r
    )
    return h, ge


# speedup vs baseline: 2.3506x; 1.6255x over previous
import dataclasses
import functools
import math

import jax
import jax.numpy as jnp
from jax import lax
from jax.experimental import pallas as pl
from jax.experimental.pallas import tpu as pltpu
from jax.experimental.pallas import tpu_sc as plsc

NC = 2
NS = 16
NT = NC * NS
CHUNK = 128


def _cdiv(a, b):
    return (a + b - 1) // b


def _mesh():
    return plsc.VectorSubcoreMesh(
        core_axis_name="c", subcore_axis_name="s", num_cores=NC, num_subcores=NS
    )


@functools.lru_cache(maxsize=None)
def _deg_kernel(K, ZR):
    cp = pltpu.CompilerParams()
    if "needs_layout_passes" in pltpu.CompilerParams.__dataclass_fields__:
        cp = dataclasses.replace(cp, needs_layout_passes=False)

    @functools.partial(
        pl.kernel,
        out_type=jax.ShapeDtypeStruct((NT, ZR), jnp.float32),
        mesh=_mesh(),
        compiler_params=cp,
        scratch_types=[
            pltpu.VMEM((K, CHUNK), jnp.int32),
            pltpu.VMEM((ZR,), jnp.float32),
        ],
    )
    def deg(dst_hbm, out_hbm, idx_v, deg_v):
        c = lax.axis_index("c")
        s = lax.axis_index("s")
        tile = s * NC + c

        @pl.loop(0, ZR, step=16)
        def _(i):
            deg_v[pl.ds(i, 16)] = jnp.zeros((16,), jnp.float32)

        pltpu.sync_copy(dst_hbm.at[pl.ds(tile * K, K)], idx_v)
        ones = jnp.ones((16,), jnp.float32)

        @pl.loop(0, K)
        def _(j):
            @pl.loop(0, CHUNK, step=16)
            def _(k):
                idx16 = idx_v[j, pl.ds(k, 16)]
                plsc.addupdate_scatter(deg_v, [idx16], ones)

        pltpu.sync_copy(deg_v, out_hbm.at[tile])

    return deg


SS = 2
IG = 16


@functools.lru_cache(maxsize=None)
def _scatter_kernel(K2, ZR, N, D):
    RPT = ZR // NS
    Dh = D // 2

    @functools.partial(
        pl.kernel,
        out_type=jax.ShapeDtypeStruct((NC, ZR, Dh), jnp.float32),
        mesh=_mesh(),
        compiler_params=pltpu.CompilerParams(use_tc_tiling_on_sc=False),
        scratch_types=[
            pltpu.VMEM((IG, CHUNK), jnp.int32),
            pltpu.VMEM((IG, CHUNK), jnp.int32),
            pltpu.VMEM((SS * CHUNK, Dh), jnp.float32),
            pltpu.VMEM((SS * CHUNK, Dh), jnp.float32),
            pltpu.SemaphoreType.DMA,
            pltpu.SemaphoreType.DMA,
            pltpu.SemaphoreType.DMA,
            pltpu.SemaphoreType.DMA,
            pltpu.VMEM_SHARED((ZR, Dh), jnp.float32),
            pltpu.VMEM_SHARED((ZR, Dh), jnp.float32),
        ],
    )
    def scat(y2_hbm, src_hbm, dst_hbm, zeros_hbm, out_hbm,
             si_v, di_v, big0, big1, gs0, gs1, ss0, ss1, z_sh, y_sh):
        big = (big0, big1)
        gsem = (gs0, gs1)
        ssem = (ss0, ss1)
        c = lax.axis_index("c")
        s = lax.axis_index("s")

        pltpu.sync_copy(
            y2_hbm.at[c, pl.ds(s * RPT, RPT)], y_sh.at[pl.ds(s * RPT, RPT)]
        )
        pltpu.sync_copy(zeros_hbm, z_sh.at[pl.ds(s * RPT, RPT)])
        plsc.subcore_barrier()
        ysrc = y_sh

        nsup = IG // SS

        def issue_gathers(u, b):
            return [
                pltpu.async_copy(
                    ysrc.at[si_v.at[u * SS + k]],
                    big[b].at[pl.ds(k * CHUNK, CHUNK)],
                    gsem[b],
                )
                for k in range(SS)
            ]

        def issue_scatters(u, b):
            return [
                pltpu.async_copy(
                    big[b].at[pl.ds(k * CHUNK, CHUNK)],
                    z_sh.at[di_v.at[u * SS + k]],
                    ssem[b],
                    add=True,
                )
                for k in range(SS)
            ]

        @pl.loop(0, K2, step=IG)
        def _(g):
            base = s * K2 + g
            pltpu.sync_copy(src_hbm.at[pl.ds(base, IG)], si_v)
            pltpu.sync_copy(dst_hbm.at[pl.ds(base, IG)], di_v)
            gd = {}
            sd = {}
            for u in range(nsup):
                b = u & 1
                if b in sd:
                    for dsc in sd.pop(b):
                        dsc.wait()
                gd[b] = issue_gathers(u, b)
                if u >= 1:
                    bb = (u - 1) & 1
                    for dsc in gd.pop(bb):
                        dsc.wait()
                    sd[bb] = issue_scatters(u - 1, bb)
            bb = (nsup - 1) & 1
            for dsc in gd.pop(bb):
                dsc.wait()
            sd[bb] = issue_scatters(nsup - 1, bb)
            for b in (0, 1):
                if b in sd:
                    for dsc in sd.pop(b):
                        dsc.wait()

        plsc.subcore_barrier()
        pltpu.sync_copy(
            z_sh.at[pl.ds(s * RPT, RPT)], out_hbm.at[c, pl.ds(s * RPT, RPT)]
        )

    return scat


def _r_body(dp_ref, o_ref):
    o_ref[...] = lax.rsqrt(jnp.sum(dp_ref[...], axis=0, keepdims=True) + 1.0)


def _split(xw, o_ref):
    dh = xw.shape[1] // 2
    pad = jnp.zeros((o_ref.shape[1] - xw.shape[0], dh), xw.dtype)
    o_ref[0] = jnp.concatenate([xw[:, :dh], pad], axis=0)
    o_ref[1] = jnp.concatenate([xw[:, dh:], pad], axis=0)


def _a_body(x_ref, w_ref, dis_ref, o_ref):
    xw = jnp.dot(x_ref[...], w_ref[...], preferred_element_type=jnp.float32)
    _split(xw * dis_ref[...], o_ref)


def _norm(z0, z1, y2, dis, b, g, be):
    n = z0.shape[0]
    pre = jnp.concatenate([z0 + y2[0, :n], z1 + y2[1, :n]], axis=1) * dis + b
    mu = jnp.mean(pre, axis=0, keepdims=True)
    ctr = pre - mu
    var = jnp.mean(ctr * ctr, axis=0, keepdims=True)
    return jnp.maximum(ctr * lax.rsqrt(var + 1e-5) * g + be, 0.0)


def _f_body(z0_ref, z1_ref, y_ref, dis_ref, b_ref, g_ref, be_ref, w_ref, o_ref):
    h = _norm(z0_ref[...], z1_ref[...], y_ref[...], dis_ref[...], b_ref[...],
              g_ref[...], be_ref[...])
    xw = jnp.dot(h, w_ref[...], preferred_element_type=jnp.float32)
    _split(xw * dis_ref[...], o_ref)


def _g_body(z0_ref, z1_ref, y_ref, dis_ref, b_ref, g_ref, be_ref, h_ref, ge_ref):
    h = _norm(z0_ref[...], z1_ref[...], y_ref[...], dis_ref[...], b_ref[...],
              g_ref[...], be_ref[...])
    h_ref[...] = h
    ge_ref[...] = jnp.mean(h, axis=0, keepdims=True)


def kernel(x, edge_index, W0, b0, g0, be0, W1, b1, g1, be1, W2, b2, g2, be2):
    N, D = x.shape
    Dh = D // 2
    E = edge_index.shape[1]
    chm = math.lcm(NS * IG, NT * 8)
    CH = _cdiv(_cdiv(E, CHUNK), chm) * chm
    K2 = CH // NS
    E_pad = CH * CHUNK
    ZR = _cdiv(N + 1, NS * 8) * NS * 8
    RPT = ZR // NS
    dummy = jnp.int32(N)

    src = edge_index[0]
    dst = edge_index[1]
    pad = E_pad - E
    src_p = jnp.concatenate([src, jnp.zeros((pad,), src.dtype)]).reshape(CH, CHUNK)
    dst_p = jnp.concatenate([dst, jnp.full((pad,), dummy, dst.dtype)]).reshape(
        CH, CHUNK
    )
    zeros_t = jnp.zeros((RPT, Dh), jnp.float32)

    degparts = _deg_kernel(CH // NT, ZR)(dst_p)
    dis_row = pl.pallas_call(
        _r_body, out_shape=jax.ShapeDtypeStruct((1, ZR), jnp.float32)
    )(degparts)
    dis_col = dis_row[0, :N].reshape(N, 1)

    f32 = jnp.float32
    nd = jax.ShapeDtypeStruct((N, D), f32)
    nd2 = jax.ShapeDtypeStruct((NC, ZR, Dh), f32)
    row = jax.ShapeDtypeStruct((1, D), f32)
    scat = _scatter_kernel(K2, ZR, N, D)
    b0r, g0r, be0r = b0.reshape(1, D), g0.reshape(1, D), be0.reshape(1, D)
    b1r, g1r, be1r = b1.reshape(1, D), g1.reshape(1, D), be1.reshape(1, D)
    b2r, g2r, be2r = b2.reshape(1, D), g2.reshape(1, D), be2.reshape(1, D)

    y2 = pl.pallas_call(_a_body, out_shape=nd2)(x, W0, dis_col)

    zraw = scat(y2, src_p, dst_p, zeros_t)
    y2 = pl.pallas_call(_f_body, out_shape=nd2)(
        zraw[0, :N], zraw[1, :N], y2, dis_col, b0r, g0r, be0r, W1
    )

    zraw = scat(y2, src_p, dst_p, zeros_t)
    y2 = pl.pallas_call(_f_body, out_shape=nd2)(
        zraw[0, :N], zraw[1, :N], y2, dis_col, b1r, g1r, be1r, W2
    )

    zraw = scat(y2, src_p, dst_p, zeros_t)
    h, ge = pl.pallas_call(_g_body, out_shape=(nd, row))(
        zraw[0, :N], zraw[1, :N], y2, dis_col, b2r, g2r, be2r
    )
    return h, ge


# speedup vs baseline: 2.4612x; 1.0471x over previous
import dataclasses
import functools
import math

import jax
import jax.numpy as jnp
from jax import lax
from jax.experimental import pallas as pl
from jax.experimental.pallas import tpu as pltpu
from jax.experimental.pallas import tpu_sc as plsc

NC = 2
NS = 16
NT = NC * NS
CHUNK = 128


def _cdiv(a, b):
    return (a + b - 1) // b


def _mesh():
    return plsc.VectorSubcoreMesh(
        core_axis_name="c", subcore_axis_name="s", num_cores=NC, num_subcores=NS
    )


@functools.lru_cache(maxsize=None)
def _deg_kernel(K, ZR):
    cp = pltpu.CompilerParams()
    if "needs_layout_passes" in pltpu.CompilerParams.__dataclass_fields__:
        cp = dataclasses.replace(cp, needs_layout_passes=False)

    @functools.partial(
        pl.kernel,
        out_type=jax.ShapeDtypeStruct((NT, ZR), jnp.float32),
        mesh=_mesh(),
        compiler_params=cp,
        scratch_types=[
            pltpu.VMEM((K, CHUNK), jnp.int32),
            pltpu.VMEM((ZR,), jnp.float32),
        ],
    )
    def deg(dst_hbm, out_hbm, idx_v, deg_v):
        c = lax.axis_index("c")
        s = lax.axis_index("s")
        tile = s * NC + c

        @pl.loop(0, ZR, step=16)
        def _(i):
            deg_v[pl.ds(i, 16)] = jnp.zeros((16,), jnp.float32)

        pltpu.sync_copy(dst_hbm.at[pl.ds(tile * K, K)], idx_v)
        ones = jnp.ones((16,), jnp.float32)

        @pl.loop(0, K)
        def _(j):
            @pl.loop(0, CHUNK, step=16)
            def _(k):
                idx16 = idx_v[j, pl.ds(k, 16)]
                plsc.addupdate_scatter(deg_v, [idx16], ones)

        pltpu.sync_copy(deg_v, out_hbm.at[tile])

    return deg


SS = 2
IG = 40


@functools.lru_cache(maxsize=None)
def _scatter_kernel(K2, ZR, N, D):
    RPT = ZR // NS
    Dh = D // 2

    @functools.partial(
        pl.kernel,
        out_type=jax.ShapeDtypeStruct((NC, ZR, Dh), jnp.float32),
        mesh=_mesh(),
        compiler_params=pltpu.CompilerParams(use_tc_tiling_on_sc=False),
        scratch_types=[
            pltpu.VMEM((IG, CHUNK), jnp.int32),
            pltpu.VMEM((IG, CHUNK), jnp.int32),
            pltpu.VMEM((SS * CHUNK, Dh), jnp.float32),
            pltpu.VMEM((SS * CHUNK, Dh), jnp.float32),
            pltpu.SemaphoreType.DMA,
            pltpu.SemaphoreType.DMA,
            pltpu.SemaphoreType.DMA,
            pltpu.SemaphoreType.DMA,
            pltpu.VMEM_SHARED((ZR, Dh), jnp.float32),
            pltpu.VMEM_SHARED((ZR, Dh), jnp.float32),
        ],
    )
    def scat(y2_hbm, src_hbm, dst_hbm, zeros_hbm, out_hbm,
             si_v, di_v, big0, big1, gs0, gs1, ss0, ss1, z_sh, y_sh):
        big = (big0, big1)
        gsem = (gs0, gs1)
        ssem = (ss0, ss1)
        c = lax.axis_index("c")
        s = lax.axis_index("s")

        pltpu.async_copy(
            y2_hbm.at[c, pl.ds(s * RPT, RPT)], y_sh.at[pl.ds(s * RPT, RPT)], gs0
        )
        pltpu.async_copy(zeros_hbm, z_sh.at[pl.ds(s * RPT, RPT)], gs1)
        pltpu.make_async_copy(
            y2_hbm.at[c, pl.ds(s * RPT, RPT)], y_sh.at[pl.ds(s * RPT, RPT)], gs0
        ).wait()
        pltpu.make_async_copy(
            zeros_hbm, z_sh.at[pl.ds(s * RPT, RPT)], gs1
        ).wait()
        plsc.subcore_barrier()
        ysrc = y_sh

        nsup = IG // SS

        def issue_gathers(u, b):
            return [
                pltpu.async_copy(
                    ysrc.at[si_v.at[u * SS + k]],
                    big[b].at[pl.ds(k * CHUNK, CHUNK)],
                    gsem[b],
                )
                for k in range(SS)
            ]

        def issue_scatters(u, b):
            return [
                pltpu.async_copy(
                    big[b].at[pl.ds(k * CHUNK, CHUNK)],
                    z_sh.at[di_v.at[u * SS + k]],
                    ssem[b],
                    add=True,
                )
                for k in range(SS)
            ]

        @pl.loop(0, K2, step=IG)
        def _(g):
            base = s * K2 + g
            pltpu.sync_copy(src_hbm.at[pl.ds(base, IG)], si_v)
            pltpu.sync_copy(dst_hbm.at[pl.ds(base, IG)], di_v)
            gd = {}
            sd = {}
            for u in range(nsup):
                b = u & 1
                if b in sd:
                    for dsc in sd.pop(b):
                        dsc.wait()
                gd[b] = issue_gathers(u, b)
                if u >= 1:
                    bb = (u - 1) & 1
                    for dsc in gd.pop(bb):
                        dsc.wait()
                    sd[bb] = issue_scatters(u - 1, bb)
            bb = (nsup - 1) & 1
            for dsc in gd.pop(bb):
                dsc.wait()
            sd[bb] = issue_scatters(nsup - 1, bb)
            for b in (0, 1):
                if b in sd:
                    for dsc in sd.pop(b):
                        dsc.wait()

        plsc.subcore_barrier()
        pltpu.sync_copy(
            z_sh.at[pl.ds(s * RPT, RPT)], out_hbm.at[c, pl.ds(s * RPT, RPT)]
        )

    return scat


def _r_body(dp_ref, o_ref):
    o_ref[...] = lax.rsqrt(jnp.sum(dp_ref[...], axis=0, keepdims=True) + 1.0)


def _a0_body(x_ref, w_ref, dp_ref, o_ref, dis_ref):
    n = x_ref.shape[0]
    deg = jnp.sum(dp_ref[...], axis=0) + 1.0
    dis_c = lax.rsqrt(deg[:n]).reshape(n, 1)
    dis_ref[...] = dis_c
    xw = jnp.dot(x_ref[...], w_ref[...], preferred_element_type=jnp.float32)
    _split(xw * dis_c, o_ref)


def _split(xw, o_ref):
    dh = xw.shape[1] // 2
    pad = jnp.zeros((o_ref.shape[1] - xw.shape[0], dh), xw.dtype)
    o_ref[0] = jnp.concatenate([xw[:, :dh], pad], axis=0)
    o_ref[1] = jnp.concatenate([xw[:, dh:], pad], axis=0)


def _a_body(x_ref, w_ref, dis_ref, o_ref):
    xw = jnp.dot(x_ref[...], w_ref[...], preferred_element_type=jnp.float32)
    _split(xw * dis_ref[...], o_ref)


def _norm(z0, z1, y2, dis, b, g, be):
    n = z0.shape[0]
    pre = jnp.concatenate([z0 + y2[0, :n], z1 + y2[1, :n]], axis=1) * dis + b
    mu = jnp.mean(pre, axis=0, keepdims=True)
    ctr = pre - mu
    var = jnp.mean(ctr * ctr, axis=0, keepdims=True)
    return jnp.maximum(ctr * lax.rsqrt(var + 1e-5) * g + be, 0.0)


def _f_body(z0_ref, z1_ref, y_ref, dis_ref, b_ref, g_ref, be_ref, w_ref, o_ref):
    h = _norm(z0_ref[...], z1_ref[...], y_ref[...], dis_ref[...], b_ref[...],
              g_ref[...], be_ref[...])
    xw = jnp.dot(h, w_ref[...], preferred_element_type=jnp.float32)
    _split(xw * dis_ref[...], o_ref)


def _g_body(z0_ref, z1_ref, y_ref, dis_ref, b_ref, g_ref, be_ref, h_ref, ge_ref):
    h = _norm(z0_ref[...], z1_ref[...], y_ref[...], dis_ref[...], b_ref[...],
              g_ref[...], be_ref[...])
    h_ref[...] = h
    ge_ref[...] = jnp.mean(h, axis=0, keepdims=True)


def kernel(x, edge_index, W0, b0, g0, be0, W1, b1, g1, be1, W2, b2, g2, be2):
    N, D = x.shape
    Dh = D // 2
    E = edge_index.shape[1]
    chm = math.lcm(NS * IG, NT * 8)
    CH = _cdiv(_cdiv(E, CHUNK), chm) * chm
    K2 = CH // NS
    E_pad = CH * CHUNK
    ZR = _cdiv(N + 1, NS * 8) * NS * 8
    RPT = ZR // NS
    dummy = jnp.int32(N)

    src = edge_index[0]
    dst = edge_index[1]
    pad = E_pad - E
    src_p = jnp.concatenate([src, jnp.zeros((pad,), src.dtype)]).reshape(CH, CHUNK)
    dst_p = jnp.concatenate([dst, jnp.full((pad,), dummy, dst.dtype)]).reshape(
        CH, CHUNK
    )
    zeros_t = jnp.zeros((RPT, Dh), jnp.float32)

    degparts = _deg_kernel(CH // NT, ZR)(dst_p)

    f32 = jnp.float32
    nd = jax.ShapeDtypeStruct((N, D), f32)
    nd2 = jax.ShapeDtypeStruct((NC, ZR, Dh), f32)
    row = jax.ShapeDtypeStruct((1, D), f32)
    scat = _scatter_kernel(K2, ZR, N, D)
    b0r, g0r, be0r = b0.reshape(1, D), g0.reshape(1, D), be0.reshape(1, D)
    b1r, g1r, be1r = b1.reshape(1, D), g1.reshape(1, D), be1.reshape(1, D)
    b2r, g2r, be2r = b2.reshape(1, D), g2.reshape(1, D), be2.reshape(1, D)

    y2, dis_col = pl.pallas_call(
        _a0_body, out_shape=(nd2, jax.ShapeDtypeStruct((N, 1), f32))
    )(x, W0, degparts)

    zraw = scat(y2, src_p, dst_p, zeros_t)
    y2 = pl.pallas_call(_f_body, out_shape=nd2)(
        zraw[0, :N], zraw[1, :N], y2, dis_col, b0r, g0r, be0r, W1
    )

    zraw = scat(y2, src_p, dst_p, zeros_t)
    y2 = pl.pallas_call(_f_body, out_shape=nd2)(
        zraw[0, :N], zraw[1, :N], y2, dis_col, b1r, g1r, be1r, W2
    )

    zraw = scat(y2, src_p, dst_p, zeros_t)
    h, ge = pl.pallas_call(_g_body, out_shape=(nd, row))(
        zraw[0, :N], zraw[1, :N], y2, dis_col, b2r, g2r, be2r
    )
    return h, ge


# speedup vs baseline: 2.5364x; 1.0305x over previous
import dataclasses
import functools
import math

import jax
import jax.numpy as jnp
from jax import lax
from jax.experimental import pallas as pl
from jax.experimental.pallas import tpu as pltpu
from jax.experimental.pallas import tpu_sc as plsc

NC = 2
NS = 16
NT = NC * NS
CHUNK = 128


def _cdiv(a, b):
    return (a + b - 1) // b


def _mesh():
    return plsc.VectorSubcoreMesh(
        core_axis_name="c", subcore_axis_name="s", num_cores=NC, num_subcores=NS
    )


@functools.lru_cache(maxsize=None)
def _deg_kernel(K, ZR):
    cp = pltpu.CompilerParams()
    if "needs_layout_passes" in pltpu.CompilerParams.__dataclass_fields__:
        cp = dataclasses.replace(cp, needs_layout_passes=False)

    @functools.partial(
        pl.kernel,
        out_type=jax.ShapeDtypeStruct((NT, ZR), jnp.float32),
        mesh=_mesh(),
        compiler_params=cp,
        scratch_types=[
            pltpu.VMEM((K, CHUNK), jnp.int32),
            pltpu.VMEM((ZR,), jnp.float32),
        ],
    )
    def deg(dst_hbm, out_hbm, idx_v, deg_v):
        c = lax.axis_index("c")
        s = lax.axis_index("s")
        tile = s * NC + c

        @pl.loop(0, ZR, step=16)
        def _(i):
            deg_v[pl.ds(i, 16)] = jnp.zeros((16,), jnp.float32)

        pltpu.sync_copy(dst_hbm.at[pl.ds(tile * K, K)], idx_v)
        ones = jnp.ones((16,), jnp.float32)

        @pl.loop(0, K)
        def _(j):
            @pl.loop(0, CHUNK, step=16)
            def _(k):
                idx16 = idx_v[j, pl.ds(k, 16)]
                plsc.addupdate_scatter(deg_v, [idx16], ones)

        pltpu.sync_copy(deg_v, out_hbm.at[tile])

    return deg


SS = 2
IG = 20


@functools.lru_cache(maxsize=None)
def _scatter_kernel(K2, ZR, N, D):
    RPT = ZR // NS
    Dh = D // 2

    @functools.partial(
        pl.kernel,
        out_type=jax.ShapeDtypeStruct((NC, ZR, Dh), jnp.float32),
        mesh=_mesh(),
        compiler_params=pltpu.CompilerParams(use_tc_tiling_on_sc=False),
        scratch_types=[
            pltpu.VMEM((2 * IG, CHUNK), jnp.int32),
            pltpu.VMEM((2 * IG, CHUNK), jnp.int32),
            pltpu.VMEM((SS * CHUNK, Dh), jnp.float32),
            pltpu.VMEM((SS * CHUNK, Dh), jnp.float32),
            pltpu.SemaphoreType.DMA,
            pltpu.SemaphoreType.DMA,
            pltpu.SemaphoreType.DMA,
            pltpu.SemaphoreType.DMA,
            pltpu.SemaphoreType.DMA,
            pltpu.SemaphoreType.DMA,
            pltpu.VMEM_SHARED((ZR, Dh), jnp.float32),
            pltpu.VMEM_SHARED((ZR, Dh), jnp.float32),
        ],
    )
    def scat(y2_hbm, ids_hbm, zeros_hbm, out_hbm,
             idx0, idx1, big0, big1, gs0, gs1, ss0, ss1, is0, is1, z_sh, y_sh):
        idxs = (idx0, idx1)
        big = (big0, big1)
        gsem = (gs0, gs1)
        ssem = (ss0, ss1)
        isem = (is0, is1)
        c = lax.axis_index("c")
        s = lax.axis_index("s")

        pltpu.async_copy(
            y2_hbm.at[c, pl.ds(s * RPT, RPT)], y_sh.at[pl.ds(s * RPT, RPT)], gs0
        )
        pltpu.async_copy(zeros_hbm, z_sh.at[pl.ds(s * RPT, RPT)], gs1)
        pltpu.make_async_copy(
            y2_hbm.at[c, pl.ds(s * RPT, RPT)], y_sh.at[pl.ds(s * RPT, RPT)], gs0
        ).wait()
        pltpu.make_async_copy(
            zeros_hbm, z_sh.at[pl.ds(s * RPT, RPT)], gs1
        ).wait()
        plsc.subcore_barrier()
        ysrc = y_sh

        nsup = IG // SS
        nsup2 = 2 * nsup
        base2 = 2 * s * K2

        def load_idx(goff, hb):
            pltpu.async_copy(
                ids_hbm.at[pl.ds(base2 + 2 * goff, 2 * IG)], idxs[hb], isem[hb]
            )

        def wait_idx(hb):
            pltpu.make_async_copy(
                ids_hbm.at[pl.ds(0, 2 * IG)], idxs[hb], isem[hb]
            ).wait()

        def idxrow(uu, k, which):
            hb = (uu * SS) // IG
            cc = (uu * SS + k) % IG
            return idxs[hb].at[2 * cc + which]

        def issue_gathers(uu, b):
            return [
                pltpu.async_copy(
                    ysrc.at[idxrow(uu, k, 0)],
                    big[b].at[pl.ds(k * CHUNK, CHUNK)],
                    gsem[b],
                )
                for k in range(SS)
            ]

        def issue_scatters(uu, b):
            return [
                pltpu.async_copy(
                    big[b].at[pl.ds(k * CHUNK, CHUNK)],
                    z_sh.at[idxrow(uu, k, 1)],
                    ssem[b],
                    add=True,
                )
                for k in range(SS)
            ]

        load_idx(0, 0)
        load_idx(IG, 1)

        @pl.loop(0, K2, step=2 * IG)
        def _(g):
            gd = {}
            sd = {}
            for uu in range(nsup2):
                if uu == 0:
                    wait_idx(0)
                if uu == nsup:
                    wait_idx(1)
                b = uu & 1
                if b in sd:
                    for dsc in sd.pop(b):
                        dsc.wait()
                if uu == nsup + 1:
                    @pl.when(g + 2 * IG < K2)
                    def _():
                        load_idx(g + 2 * IG, 0)
                gd[b] = issue_gathers(uu, b)
                if uu >= 1:
                    bb = (uu - 1) & 1
                    for dsc in gd.pop(bb):
                        dsc.wait()
                    sd[bb] = issue_scatters(uu - 1, bb)
            bb = (nsup2 - 1) & 1
            for dsc in gd.pop(bb):
                dsc.wait()
            sd[bb] = issue_scatters(nsup2 - 1, bb)
            for b in (0, 1):
                if b in sd:
                    for dsc in sd.pop(b):
                        dsc.wait()

            @pl.when(g + 2 * IG < K2)
            def _():
                load_idx(g + 3 * IG, 1)

        plsc.subcore_barrier()
        pltpu.sync_copy(
            z_sh.at[pl.ds(s * RPT, RPT)], out_hbm.at[c, pl.ds(s * RPT, RPT)]
        )

    return scat


def _r_body(dp_ref, o_ref):
    o_ref[...] = lax.rsqrt(jnp.sum(dp_ref[...], axis=0, keepdims=True) + 1.0)


def _a0_body(x_ref, w_ref, dp_ref, o_ref, dis_ref):
    n = x_ref.shape[0]
    deg = jnp.sum(dp_ref[...], axis=0) + 1.0
    dis_c = lax.rsqrt(deg[:n]).reshape(n, 1)
    dis_ref[...] = dis_c
    xw = jnp.dot(x_ref[...], w_ref[...], preferred_element_type=jnp.float32)
    _split(xw * dis_c, o_ref)


def _split(xw, o_ref):
    dh = xw.shape[1] // 2
    pad = jnp.zeros((o_ref.shape[1] - xw.shape[0], dh), xw.dtype)
    o_ref[0] = jnp.concatenate([xw[:, :dh], pad], axis=0)
    o_ref[1] = jnp.concatenate([xw[:, dh:], pad], axis=0)


def _a_body(x_ref, w_ref, dis_ref, o_ref):
    xw = jnp.dot(x_ref[...], w_ref[...], preferred_element_type=jnp.float32)
    _split(xw * dis_ref[...], o_ref)


def _norm(z0, z1, y2, dis, b, g, be):
    n = z0.shape[0]
    pre = jnp.concatenate([z0 + y2[0, :n], z1 + y2[1, :n]], axis=1) * dis + b
    mu = jnp.mean(pre, axis=0, keepdims=True)
    ctr = pre - mu
    var = jnp.mean(ctr * ctr, axis=0, keepdims=True)
    return jnp.maximum(ctr * lax.rsqrt(var + 1e-5) * g + be, 0.0)


def _f_body(z0_ref, z1_ref, y_ref, dis_ref, b_ref, g_ref, be_ref, w_ref, o_ref):
    h = _norm(z0_ref[...], z1_ref[...], y_ref[...], dis_ref[...], b_ref[...],
              g_ref[...], be_ref[...])
    xw = jnp.dot(h, w_ref[...], preferred_element_type=jnp.float32)
    _split(xw * dis_ref[...], o_ref)


def _g_body(z0_ref, z1_ref, y_ref, dis_ref, b_ref, g_ref, be_ref, h_ref, ge_ref):
    h = _norm(z0_ref[...], z1_ref[...], y_ref[...], dis_ref[...], b_ref[...],
              g_ref[...], be_ref[...])
    h_ref[...] = h
    ge_ref[...] = jnp.mean(h, axis=0, keepdims=True)


def kernel(x, edge_index, W0, b0, g0, be0, W1, b1, g1, be1, W2, b2, g2, be2):
    N, D = x.shape
    Dh = D // 2
    E = edge_index.shape[1]
    chm = math.lcm(NS * IG, NT * 8)
    CH = _cdiv(_cdiv(E, CHUNK), chm) * chm
    K2 = CH // NS
    E_pad = CH * CHUNK
    ZR = _cdiv(N + 1, NS * 8) * NS * 8
    RPT = ZR // NS
    dummy = jnp.int32(N)

    src = edge_index[0]
    dst = edge_index[1]
    pad = E_pad - E
    src_p = jnp.concatenate([src, jnp.zeros((pad,), src.dtype)]).reshape(CH, CHUNK)
    dst_p = jnp.concatenate([dst, jnp.full((pad,), dummy, dst.dtype)]).reshape(
        CH, CHUNK
    )
    ids = jnp.concatenate(
        [
            jnp.stack([src_p, dst_p], axis=1).reshape(2 * CH, CHUNK),
            jnp.zeros((4 * IG, CHUNK), src.dtype),
        ]
    )
    zeros_t = jnp.zeros((RPT, Dh), jnp.float32)

    degparts = _deg_kernel(CH // NT, ZR)(dst_p)

    f32 = jnp.float32
    nd = jax.ShapeDtypeStruct((N, D), f32)
    nd2 = jax.ShapeDtypeStruct((NC, ZR, Dh), f32)
    row = jax.ShapeDtypeStruct((1, D), f32)
    scat = _scatter_kernel(K2, ZR, N, D)
    b0r, g0r, be0r = b0.reshape(1, D), g0.reshape(1, D), be0.reshape(1, D)
    b1r, g1r, be1r = b1.reshape(1, D), g1.reshape(1, D), be1.reshape(1, D)
    b2r, g2r, be2r = b2.reshape(1, D), g2.reshape(1, D), be2.reshape(1, D)

    y2, dis_col = pl.pallas_call(
        _a0_body, out_shape=(nd2, jax.ShapeDtypeStruct((N, 1), f32))
    )(x, W0, degparts)

    zraw = scat(y2, ids, zeros_t)
    y2 = pl.pallas_call(_f_body, out_shape=nd2)(
        zraw[0, :N], zraw[1, :N], y2, dis_col, b0r, g0r, be0r, W1
    )

    zraw = scat(y2, ids, zeros_t)
    y2 = pl.pallas_call(_f_body, out_shape=nd2)(
        zraw[0, :N], zraw[1, :N], y2, dis_col, b1r, g1r, be1r, W2
    )

    zraw = scat(y2, ids, zeros_t)
    h, ge = pl.pallas_call(_g_body, out_shape=(nd, row))(
        zraw[0, :N], zraw[1, :N], y2, dis_col, b2r, g2r, be2r
    )
    return h, ge
